# Initial kernel scaffold; baseline (speedup 1.0000x reference)
import jax
import jax.numpy as jnp
from jax.experimental import pallas as pl


def kernel(x, edge_index, edge_weight, batch, W1, b1, W2, b2, Wh, bh):
    raise NotImplementedError("write your pallas kernel here")


# speedup vs baseline: 1.0950x; 1.0950x over previous
import jax
import jax.numpy as jnp
from jax.experimental import pallas as pl

N = 10000
G = 64


def _mm_body(a_ref, b_ref, o_ref):
    o_ref[...] = jnp.dot(a_ref[...], b_ref[...], preferred_element_type=jnp.float32)


def _mm(a, b):
    return pl.pallas_call(
        _mm_body,
        out_shape=jax.ShapeDtypeStruct((a.shape[0], b.shape[1]), jnp.float32),
    )(a, b)


def _conv(x, src, dst, ew, W, b):
    h = _mm(x, W)
    deg = jax.ops.segment_sum(ew, dst, num_segments=N)
    safe = jnp.where(deg > 0, deg, 1.0)
    dinv = jnp.where(deg > 0, 1.0 / jnp.sqrt(safe), 0.0)
    norm = dinv[src] * ew * dinv[dst]
    msg = h[src] * norm[:, None]
    out = jax.ops.segment_sum(msg, dst, num_segments=N)
    return out + b


def kernel(x, edge_index, edge_weight, batch, W1, b1, W2, b2, Wh, bh):
    src, dst = edge_index[0], edge_index[1]
    loop = jnp.arange(N, dtype=src.dtype)
    src2 = jnp.concatenate([src, loop])
    dst2 = jnp.concatenate([dst, loop])
    ew2 = jnp.concatenate([edge_weight, jnp.ones((N,), dtype=edge_weight.dtype)])
    h = jax.nn.relu(_conv(x, src2, dst2, ew2, W1, b1))
    h = jax.nn.relu(_conv(h, src2, dst2, ew2, W2, b2))
    sums = jax.ops.segment_sum(h, batch, num_segments=G)
    cnts = jax.ops.segment_sum(jnp.ones((N,), dtype=h.dtype), batch, num_segments=G)
    pooled = sums / jnp.maximum(cnts, 1.0)[:, None]
    return _mm(pooled, Wh) + bh


# speedup vs baseline: 9.0769x; 8.2893x over previous
import functools

import jax
import jax.numpy as jnp
from jax import lax
from jax.experimental import pallas as pl
from jax.experimental.pallas import tpu as pltpu, tpu_sc as plsc

N = 10000
D = 128
G = 64
E = 320000

NC = 2
NS = 16
CHUNK = 128
EP = ((E + NC * NS * CHUNK - 1) // (NC * NS * CHUNK)) * NC * NS * CHUNK
EPT = EP // (NC * NS)
NCH = EPT // CHUNK
RPT = 624
RREM = N - NS * RPT

_mesh = plsc.VectorSubcoreMesh(core_axis_name="c", subcore_axis_name="s")


def _deg_body(dst_h, ew_h, zeros_h, out_h, dstv, ewv, zv, acc):
    cid = lax.axis_index("c")
    sid = lax.axis_index("s")
    pltpu.sync_copy(zeros_h.at[pl.ds(0, RPT)], zv)
    pltpu.sync_copy(zv, acc.at[pl.ds(sid * RPT, RPT)])

    @pl.when(sid == NS - 1)
    def _():
        pltpu.sync_copy(zv.at[pl.ds(0, RREM)], acc.at[pl.ds(NS * RPT, RREM)])

    plsc.subcore_barrier()
    tile_base = cid * (EP // 2) + sid * EPT

    def chunk(i, carry):
        base = tile_base + i * CHUNK
        pltpu.sync_copy(dst_h.at[pl.ds(base, CHUNK)], dstv)
        pltpu.sync_copy(ew_h.at[pl.ds(base, CHUNK)], ewv)
        pltpu.sync_copy(ewv, acc.at[dstv], add=True)
        return carry

    lax.fori_loop(0, NCH, chunk, 0)
    plsc.subcore_barrier()
    pltpu.sync_copy(acc.at[pl.ds(sid * RPT, RPT)], zv)
    pltpu.sync_copy(zv, out_h.at[pl.ds(cid * N + sid * RPT, RPT)])

    @pl.when(sid == NS - 1)
    def _():
        pltpu.sync_copy(acc.at[pl.ds(NS * RPT, RREM)], zv.at[pl.ds(0, RREM)])
        pltpu.sync_copy(zv.at[pl.ds(0, RREM)], out_h.at[pl.ds(cid * N + NS * RPT, RREM)])


_deg_kernel = functools.partial(
    pl.kernel,
    out_type=jax.ShapeDtypeStruct((NC * N,), jnp.float32),
    mesh=_mesh,
    scratch_types=[
        pltpu.VMEM((CHUNK,), jnp.int32),
        pltpu.VMEM((CHUNK,), jnp.float32),
        pltpu.VMEM((RPT,), jnp.float32),
        pltpu.VMEM_SHARED((N,), jnp.float32),
    ],
)(_deg_body)


def _make_agg():
    pieces = [(0, 128), (128, 128), (256, 128), (384, 128), (512, 112)]

    def body(g_h, src_h, dst_h, ew_h, zeros_h, out_h, srcv, dstv, rows, ewv, acc):
        cid = lax.axis_index("c")
        sid = lax.axis_index("s")
        pltpu.sync_copy(zeros_h.at[pl.ds(0, CHUNK)], rows)
        for off, size in pieces:
            pltpu.sync_copy(rows.at[pl.ds(0, size)],
                            acc.at[pl.ds(sid * RPT + off, size)])

        @pl.when(sid == NS - 1)
        def _():
            pltpu.sync_copy(rows.at[pl.ds(0, RREM)], acc.at[pl.ds(NS * RPT, RREM)])

        plsc.subcore_barrier()
        tile_base = cid * (EP // 2) + sid * EPT

        def chunk(i, carry):
            base = tile_base + i * CHUNK
            pltpu.sync_copy(src_h.at[pl.ds(base, CHUNK)], srcv)
            pltpu.sync_copy(dst_h.at[pl.ds(base, CHUNK)], dstv)
            pltpu.sync_copy(ew_h.at[pl.ds(base, CHUNK)], ewv)
            pltpu.sync_copy(g_h.at[srcv], rows)

            def group(k16, c2):
                ewvec = ewv[pl.ds(k16 * 16, 16)]
                for t in range(16):
                    s = lax.gather(
                        ewvec, jnp.full((16, 1), t, jnp.int32),
                        lax.GatherDimensionNumbers(
                            offset_dims=(), collapsed_slice_dims=(0,),
                            start_index_map=(0,)),
                        slice_sizes=(1,),
                        mode=lax.GatherScatterMode.PROMISE_IN_BOUNDS)
                    k = k16 * 16 + t
                    for j in range(D // 16):
                        sl = pl.ds(j * 16, 16)
                        rows[k, sl] = rows[k, sl] * s
                return c2

            lax.fori_loop(0, CHUNK // 16, group, 0)
            pltpu.sync_copy(rows, acc.at[dstv], add=True)
            return carry

        lax.fori_loop(0, NCH, chunk, 0)
        plsc.subcore_barrier()
        for off, size in pieces:
            pltpu.sync_copy(acc.at[pl.ds(sid * RPT + off, size)],
                            rows.at[pl.ds(0, size)])
            pltpu.sync_copy(rows.at[pl.ds(0, size)],
                            out_h.at[cid, pl.ds(sid * RPT + off, size)])

        @pl.when(sid == NS - 1)
        def _():
            pltpu.sync_copy(acc.at[pl.ds(NS * RPT, RREM)], rows.at[pl.ds(0, RREM)])
            pltpu.sync_copy(rows.at[pl.ds(0, RREM)], out_h.at[cid, pl.ds(NS * RPT, RREM)])

    return pl.kernel(
        body,
        out_type=jax.ShapeDtypeStruct((NC, N, D), jnp.float32),
        mesh=_mesh,
        scratch_types=[
            pltpu.VMEM((CHUNK,), jnp.int32),
            pltpu.VMEM((CHUNK,), jnp.int32),
            pltpu.VMEM((CHUNK, D), jnp.float32),
            pltpu.VMEM((CHUNK,), jnp.float32),
            pltpu.VMEM_SHARED((N, D), jnp.float32),
        ],
    )


_agg_kernel = _make_agg()


def _g1_body(x_ref, w_ref, d0_ref, d1_ref, o_ref):
    deg = d0_ref[...] + d1_ref[...] + 1.0
    dinv = lax.rsqrt(deg)
    o_ref[...] = jnp.dot(x_ref[...], w_ref[...], preferred_element_type=jnp.float32) * dinv


def _layer_body(a0_ref, a1_ref, g_ref, d0_ref, d1_ref, b_ref, w_ref, o_ref):
    deg = d0_ref[...] + d1_ref[...] + 1.0
    dinv = lax.rsqrt(deg)
    h = jnp.maximum(dinv * (a0_ref[...] + a1_ref[...] + g_ref[...]) + b_ref[...], 0.0)
    o_ref[...] = jnp.dot(h, w_ref[...], preferred_element_type=jnp.float32) * dinv


def _final_body(a0_ref, a1_ref, g_ref, d0_ref, d1_ref, b_ref, wh_ref, bh_ref,
                batch_ref, o_ref):
    deg = d0_ref[...] + d1_ref[...] + 1.0
    dinv = lax.rsqrt(deg)
    h = jnp.maximum(dinv * (a0_ref[...] + a1_ref[...] + g_ref[...]) + b_ref[...], 0.0)
    iota = lax.broadcasted_iota(jnp.int32, (G, N), 0)
    mask = (batch_ref[...] == iota).astype(jnp.float32)
    sums = jnp.dot(mask, h, preferred_element_type=jnp.float32)
    cnts = jnp.sum(mask, axis=1, keepdims=True)
    pooled = sums / jnp.maximum(cnts, 1.0)
    o_ref[...] = jnp.dot(pooled, wh_ref[...], preferred_element_type=jnp.float32) + bh_ref[...]


def _tc_g1(x, w, d0, d1):
    return pl.pallas_call(
        _g1_body, out_shape=jax.ShapeDtypeStruct((N, D), jnp.float32),
    )(x, w, d0, d1)


def _tc_layer(a0, a1, g, d0, d1, b, w):
    return pl.pallas_call(
        _layer_body, out_shape=jax.ShapeDtypeStruct((N, D), jnp.float32),
    )(a0, a1, g, d0, d1, b, w)


def _tc_final(a0, a1, g, d0, d1, b, wh, bh, batch):
    return pl.pallas_call(
        _final_body, out_shape=jax.ShapeDtypeStruct((G, 1), jnp.float32),
    )(a0, a1, g, d0, d1, b, wh, bh, batch)


def kernel(x, edge_index, edge_weight, batch, W1, b1, W2, b2, Wh, bh):
    src = edge_index[0]
    dst = edge_index[1]
    pad = EP - E
    zi = jnp.zeros((pad,), jnp.int32)
    src_p = jnp.concatenate([src, zi])
    dst_p = jnp.concatenate([dst, zi])
    ew_p = jnp.concatenate([edge_weight, jnp.zeros((pad,), jnp.float32)])
    zeros_rows = jnp.zeros((N, D), jnp.float32)
    zeros_n = jnp.zeros((N,), jnp.float32)

    dparts = _deg_kernel(dst_p, ew_p, zeros_n)
    d0 = dparts[:N].reshape(N, 1)
    d1 = dparts[N:].reshape(N, 1)

    g1 = _tc_g1(x, W1, d0, d1)
    a1 = _agg_kernel(g1, src_p, dst_p, ew_p, zeros_rows)
    g2 = _tc_layer(a1[0], a1[1], g1, d0, d1, b1.reshape(1, D), W2)
    a2 = _agg_kernel(g2, src_p, dst_p, ew_p, zeros_rows)
    return _tc_final(a2[0], a2[1], g2, d0, d1, b2.reshape(1, D), Wh,
                     bh.reshape(1, 1), batch.reshape(1, N))
